# baseline (speedup 1.0000x reference)
import jax
import jax.numpy as jnp
from jax import lax
from jax.experimental import pallas as pl
from jax.experimental.pallas import tpu as pltpu
from jax.experimental.pallas import tpu_sc as plsc

N = 1_000_000
NC = 1
NS = 16
L = 16
NW = NC * NS

ROWS = 7812
BR = 432
GRID = 17
TC_ROWS = BR * GRID
SC_ROWS = ROWS - TC_ROWS
SC_START = TC_ROWS * 128
CHUNK = SC_ROWS * 128 // NW
NVEC = CHUNK // L
UNROLL = 6
NMAIN = NVEC // UNROLL
TAIL_BASE = ROWS * 128
TAIL_ELEMS = N - TAIL_BASE
TAIL_VEC = TAIL_ELEMS // L
CAND_ROWS = NW * L // 128
BIG_I32 = 1 << 30

_LOG_P = (
    7.0376836292e-2, -1.1514610310e-1, 1.1676998740e-1,
    -1.2420140846e-1, 1.4249322787e-1, -1.6668057665e-1,
    2.0000714765e-1, -2.4999993993e-1, 3.3333331174e-1,
)
_LN2_HI = 0.693359375
_LN2_LO = -2.12194440e-4
_SQRT_HALF_BITS = 0x3F3504F3


def _log(u):
    bits = lax.bitcast_convert_type(u, jnp.int32)
    e = lax.shift_right_arithmetic(bits - _SQRT_HALF_BITS, 23)
    m = lax.bitcast_convert_type(bits - lax.shift_left(e, 23), jnp.float32)
    f = m - 1.0
    ef = e.astype(jnp.float32)
    z = f * f
    poly = jnp.float32(_LOG_P[0])
    for c in _LOG_P[1:]:
        poly = poly * f + c
    y = poly * f * z + ef * _LN2_LO - 0.5 * z
    return (f + y) + ef * _LN2_HI


def _score(u, p):
    return _log(u) * jnp.exp(-p)


def _upd(xb, ib, x, idx):
    take = x > xb
    return jnp.where(take, x, xb), jnp.where(take, idx, ib)


def _merge(xa, ia, xb, ibv):
    take = (xb > xa) | ((xb == xa) & (ibv < ia))
    return jnp.where(take, xb, xa), jnp.where(take, ibv, ia)


def _sc_body(p_hbm, u_hbm, x_out, i_out, u_v, p_v, x_s, i_s):
    c = lax.axis_index("c")
    s = lax.axis_index("s")
    wid = c * NS + s
    base = SC_START + wid * CHUNK
    pltpu.sync_copy(u_hbm.at[pl.ds(base, CHUNK)], u_v.at[pl.ds(0, CHUNK)])
    pltpu.sync_copy(p_hbm.at[pl.ds(base, CHUNK)], p_v.at[pl.ds(0, CHUNK)])
    pltpu.sync_copy(u_hbm.at[pl.ds(TAIL_BASE, TAIL_ELEMS)],
                    u_v.at[pl.ds(CHUNK, TAIL_ELEMS)])
    pltpu.sync_copy(p_hbm.at[pl.ds(TAIL_BASE, TAIL_ELEMS)],
                    p_v.at[pl.ds(CHUNK, TAIL_ELEMS)])

    iota = lax.iota(jnp.int32, L)
    neg_inf = jnp.full((L,), -jnp.inf, jnp.float32)
    big = jnp.full((L,), BIG_I32, jnp.int32)

    def step(i, carry):
        st = list(carry)
        off = i * (UNROLL * L)
        for j in range(UNROLL):
            o = off + j * L
            x = _score(u_v[pl.ds(o, L)], p_v[pl.ds(o, L)])
            st[2 * j], st[2 * j + 1] = _upd(st[2 * j], st[2 * j + 1], x,
                                            iota + (base + o))
        return tuple(st)

    st = lax.fori_loop(0, NMAIN, step, (neg_inf, big) * UNROLL)
    st = list(st)

    for k in range(TAIL_VEC):
        o = CHUNK + k * L
        x = _score(u_v[pl.ds(o, L)], p_v[pl.ds(o, L)])
        j = k % UNROLL
        st[2 * j], st[2 * j + 1] = _upd(st[2 * j], st[2 * j + 1], x,
                                        iota + (TAIL_BASE + k * L))

    xb, ib = st[0], st[1]
    for j in range(1, UNROLL):
        xb, ib = _merge(xb, ib, st[2 * j], st[2 * j + 1])

    x_s[...] = xb
    i_s[...] = ib
    pltpu.sync_copy(x_s, x_out.at[pl.ds(wid * L, L)])
    pltpu.sync_copy(i_s, i_out.at[pl.ds(wid * L, L)])


_sc_mesh = plsc.VectorSubcoreMesh(core_axis_name="c", subcore_axis_name="s",
                                  num_cores=NC, num_subcores=NS)

_sc_stage = pl.kernel(
    _sc_body,
    out_type=(jax.ShapeDtypeStruct((NW * L,), jnp.float32),
              jax.ShapeDtypeStruct((NW * L,), jnp.int32)),
    mesh=_sc_mesh,
    scratch_types=[
        pltpu.VMEM((CHUNK + TAIL_ELEMS,), jnp.float32),
        pltpu.VMEM((CHUNK + TAIL_ELEMS,), jnp.float32),
        pltpu.VMEM((L,), jnp.float32),
        pltpu.VMEM((L,), jnp.int32),
    ],
)


def _tc_body(p_ref, u_ref, xc_ref, ic_ref, out_ref, acc_x, acc_b):
    i = pl.program_id(0)
    xw = jnp.log(u_ref[...]) * jnp.exp(-p_ref[...])

    @pl.when(i == 0)
    def _init():
        acc_x[...] = xw
        acc_b[...] = jnp.zeros((BR, 128), jnp.int32)

    @pl.when(i > 0)
    def _acc():
        take = xw > acc_x[...]
        acc_x[...] = jnp.where(take, xw, acc_x[...])
        acc_b[...] = jnp.where(take, jnp.full((BR, 128), 1, jnp.int32) * i,
                               acc_b[...])

    @pl.when(i == GRID - 1)
    def _finish():
        ax = acc_x[...]
        ab = acc_b[...]
        rows = lax.broadcasted_iota(jnp.int32, (BR, 128), 0)
        cols = lax.broadcasted_iota(jnp.int32, (BR, 128), 1)
        gidx = (ab * BR + rows) * 128 + cols
        m_tc = jnp.max(ax)
        i_tc = jnp.min(jnp.where(ax == m_tc, gidx, BIG_I32))
        xc = xc_ref[...]
        ic = ic_ref[...]
        m_sc = jnp.max(xc)
        i_sc = jnp.min(jnp.where(xc == m_sc, ic, BIG_I32))
        take_sc = (m_sc > m_tc) | ((m_sc == m_tc) & (i_sc < i_tc))
        out_ref[0, 0] = jnp.where(take_sc, i_sc, i_tc)


_tc_stage = pl.pallas_call(
    _tc_body,
    grid=(GRID,),
    in_specs=[
        pl.BlockSpec((BR, 128), lambda i: (i, 0)),
        pl.BlockSpec((BR, 128), lambda i: (i, 0)),
        pl.BlockSpec((CAND_ROWS, 128), lambda i: (0, 0)),
        pl.BlockSpec((CAND_ROWS, 128), lambda i: (0, 0)),
    ],
    out_specs=pl.BlockSpec(memory_space=pltpu.SMEM),
    out_shape=jax.ShapeDtypeStruct((1, 1), jnp.int32),
    scratch_shapes=[
        pltpu.VMEM((BR, 128), jnp.float32),
        pltpu.VMEM((BR, 128), jnp.int32),
    ],
)


def kernel(p, u):
    x_cand, i_cand = _sc_stage(p, u)
    p2 = p[:TAIL_BASE].reshape(ROWS, 128)
    u2 = u[:TAIL_BASE].reshape(ROWS, 128)
    ans = _tc_stage(p2, u2,
                    x_cand.reshape(CAND_ROWS, 128),
                    i_cand.reshape(CAND_ROWS, 128))
    return ans[0, 0]


# speedup vs baseline: 1.0654x; 1.0654x over previous
import jax
import jax.numpy as jnp
from jax import lax
from jax.experimental import pallas as pl
from jax.experimental.pallas import tpu as pltpu
from jax.experimental.pallas import tpu_sc as plsc

N = 1_000_000
NC = 1
NS = 16
L = 16
NW = NC * NS

ROWS = 7812
BR = 432
GRID = 17
TC_ROWS = BR * GRID
SC_ROWS = ROWS - TC_ROWS
SC_START = TC_ROWS * 128
CHUNK = SC_ROWS * 128 // NW
NVEC = CHUNK // L
UNROLL = 6
NMAIN = NVEC // UNROLL
TAIL_BASE = ROWS * 128
TAIL_ELEMS = N - TAIL_BASE
TAIL_VEC = TAIL_ELEMS // L
CAND_ROWS = NW * L // 128
BIG_I32 = 1 << 30

_LOG_P = (
    7.0376836292e-2, -1.1514610310e-1, 1.1676998740e-1,
    -1.2420140846e-1, 1.4249322787e-1, -1.6668057665e-1,
    2.0000714765e-1, -2.4999993993e-1, 3.3333331174e-1,
)
_LN2_HI = 0.693359375
_LN2_LO = -2.12194440e-4
_SQRT_HALF_BITS = 0x3F3504F3


def _log(u):
    bits = lax.bitcast_convert_type(u, jnp.int32)
    e = lax.shift_right_arithmetic(bits - _SQRT_HALF_BITS, 23)
    m = lax.bitcast_convert_type(bits - lax.shift_left(e, 23), jnp.float32)
    f = m - 1.0
    ef = e.astype(jnp.float32)
    z = f * f
    poly = jnp.float32(_LOG_P[0])
    for c in _LOG_P[1:]:
        poly = poly * f + c
    y = poly * f * z + ef * _LN2_LO - 0.5 * z
    return (f + y) + ef * _LN2_HI


def _score(u, p):
    return _log(u) * jnp.exp(-p)


def _upd(xb, ib, x, idx):
    take = x > xb
    return jnp.where(take, x, xb), jnp.where(take, idx, ib)


def _merge(xa, ia, xb, ibv):
    take = (xb > xa) | ((xb == xa) & (ibv < ia))
    return jnp.where(take, xb, xa), jnp.where(take, ibv, ia)


def _sc_body(p_hbm, u_hbm, x_out, i_out, u_v, p_v, x_s, i_s, dsem):
    c = lax.axis_index("c")
    s = lax.axis_index("s")
    wid = c * NS + s
    base = SC_START + wid * CHUNK
    copies = [
        pltpu.async_copy(u_hbm.at[pl.ds(base, CHUNK)],
                         u_v.at[pl.ds(0, CHUNK)], dsem),
        pltpu.async_copy(p_hbm.at[pl.ds(base, CHUNK)],
                         p_v.at[pl.ds(0, CHUNK)], dsem),
        pltpu.async_copy(u_hbm.at[pl.ds(TAIL_BASE, TAIL_ELEMS)],
                         u_v.at[pl.ds(CHUNK, TAIL_ELEMS)], dsem),
        pltpu.async_copy(p_hbm.at[pl.ds(TAIL_BASE, TAIL_ELEMS)],
                         p_v.at[pl.ds(CHUNK, TAIL_ELEMS)], dsem),
    ]
    for cpy in copies:
        cpy.wait()

    iota = lax.iota(jnp.int32, L)
    neg_inf = jnp.full((L,), -jnp.inf, jnp.float32)
    big = jnp.full((L,), BIG_I32, jnp.int32)

    def step(i, carry):
        st = list(carry)
        off = i * (UNROLL * L)
        for j in range(UNROLL):
            o = off + j * L
            x = _score(u_v[pl.ds(o, L)], p_v[pl.ds(o, L)])
            st[2 * j], st[2 * j + 1] = _upd(st[2 * j], st[2 * j + 1], x,
                                            iota + (base + o))
        return tuple(st)

    st = lax.fori_loop(0, NMAIN, step, (neg_inf, big) * UNROLL)
    st = list(st)

    for k in range(TAIL_VEC):
        o = CHUNK + k * L
        x = _score(u_v[pl.ds(o, L)], p_v[pl.ds(o, L)])
        j = k % UNROLL
        st[2 * j], st[2 * j + 1] = _upd(st[2 * j], st[2 * j + 1], x,
                                        iota + (TAIL_BASE + k * L))

    xb, ib = st[0], st[1]
    for j in range(1, UNROLL):
        xb, ib = _merge(xb, ib, st[2 * j], st[2 * j + 1])

    x_s[...] = xb
    i_s[...] = ib
    co1 = pltpu.async_copy(x_s, x_out.at[pl.ds(wid * L, L)], dsem)
    co2 = pltpu.async_copy(i_s, i_out.at[pl.ds(wid * L, L)], dsem)
    co1.wait()
    co2.wait()


_sc_mesh = plsc.VectorSubcoreMesh(core_axis_name="c", subcore_axis_name="s",
                                  num_cores=NC, num_subcores=NS)

_sc_stage = pl.kernel(
    _sc_body,
    out_type=(jax.ShapeDtypeStruct((NW * L,), jnp.float32),
              jax.ShapeDtypeStruct((NW * L,), jnp.int32)),
    mesh=_sc_mesh,
    scratch_types=[
        pltpu.VMEM((CHUNK + TAIL_ELEMS,), jnp.float32),
        pltpu.VMEM((CHUNK + TAIL_ELEMS,), jnp.float32),
        pltpu.VMEM((L,), jnp.float32),
        pltpu.VMEM((L,), jnp.int32),
        pltpu.SemaphoreType.DMA,
    ],
)


def _tc_body(p_hbm, u_hbm, xc_ref, ic_ref, out_ref,
             pb0, ub0, pb1, ub1, acc_x, acc_b, sem0, sem1):
    def copies(i, pb, ub, sem):
        return (pltpu.make_async_copy(p_hbm.at[pl.ds(i * BR, BR)], pb, sem),
                pltpu.make_async_copy(u_hbm.at[pl.ds(i * BR, BR)], ub, sem))

    for cpy in copies(0, pb0, ub0, sem0):
        cpy.start()

    def phase(i, pb, ub, sem, npb, nub, nsem):
        @pl.when(i + 1 < GRID)
        def _prefetch():
            for cpy in copies(i + 1, npb, nub, nsem):
                cpy.start()

        for cpy in copies(i, pb, ub, sem):
            cpy.wait()
        xw = jnp.log(ub[...]) * jnp.exp(-pb[...])

        @pl.when(i == 0)
        def _init():
            acc_x[...] = xw
            acc_b[...] = jnp.zeros((BR, 128), jnp.int32)

        @pl.when(i > 0)
        def _acc():
            take = xw > acc_x[...]
            acc_x[...] = jnp.where(take, xw, acc_x[...])
            acc_b[...] = jnp.where(take,
                                   jnp.broadcast_to(i, (BR, 128)),
                                   acc_b[...])

    def body(i, carry):
        @pl.when(lax.rem(i, 2) == 0)
        def _even():
            phase(i, pb0, ub0, sem0, pb1, ub1, sem1)

        @pl.when(lax.rem(i, 2) == 1)
        def _odd():
            phase(i, pb1, ub1, sem1, pb0, ub0, sem0)

        return carry

    lax.fori_loop(0, GRID, body, 0)

    ax = acc_x[...]
    ab = acc_b[...]
    rows = lax.broadcasted_iota(jnp.int32, (BR, 128), 0)
    cols = lax.broadcasted_iota(jnp.int32, (BR, 128), 1)
    gidx = (ab * BR + rows) * 128 + cols
    m_tc = jnp.max(ax)
    i_tc = jnp.min(jnp.where(ax == m_tc, gidx, BIG_I32))
    xc = xc_ref[...]
    ic = ic_ref[...]
    m_sc = jnp.max(xc)
    i_sc = jnp.min(jnp.where(xc == m_sc, ic, BIG_I32))
    take_sc = (m_sc > m_tc) | ((m_sc == m_tc) & (i_sc < i_tc))
    out_ref[0, 0] = jnp.where(take_sc, i_sc, i_tc)


_tc_stage = pl.pallas_call(
    _tc_body,
    in_specs=[
        pl.BlockSpec(memory_space=pl.ANY),
        pl.BlockSpec(memory_space=pl.ANY),
        pl.BlockSpec((CAND_ROWS, 128), lambda: (0, 0)),
        pl.BlockSpec((CAND_ROWS, 128), lambda: (0, 0)),
    ],
    out_specs=pl.BlockSpec(memory_space=pltpu.SMEM),
    out_shape=jax.ShapeDtypeStruct((1, 1), jnp.int32),
    scratch_shapes=[
        pltpu.VMEM((BR, 128), jnp.float32),
        pltpu.VMEM((BR, 128), jnp.float32),
        pltpu.VMEM((BR, 128), jnp.float32),
        pltpu.VMEM((BR, 128), jnp.float32),
        pltpu.VMEM((BR, 128), jnp.float32),
        pltpu.VMEM((BR, 128), jnp.int32),
        pltpu.SemaphoreType.DMA,
        pltpu.SemaphoreType.DMA,
    ],
)


def kernel(p, u):
    x_cand, i_cand = _sc_stage(p, u)
    p2 = p[:TAIL_BASE].reshape(ROWS, 128)
    u2 = u[:TAIL_BASE].reshape(ROWS, 128)
    ans = _tc_stage(p2, u2,
                    x_cand.reshape(CAND_ROWS, 128),
                    i_cand.reshape(CAND_ROWS, 128))
    return ans[0, 0]


# speedup vs baseline: 1.0914x; 1.0244x over previous
import jax
import jax.numpy as jnp
from jax import lax
from jax.experimental import pallas as pl
from jax.experimental.pallas import tpu as pltpu
from jax.experimental.pallas import tpu_sc as plsc

N = 1_000_000
NC = 1
NS = 16
L = 16
NW = NC * NS

ROWS = 7812
BR = 432
GRID = 17
TC_ROWS = BR * GRID
SC_ROWS = ROWS - TC_ROWS
SC_START = TC_ROWS * 128
CHUNK = SC_ROWS * 128 // NW
NVEC = CHUNK // L
UNROLL = 6
NMAIN = NVEC // UNROLL
TAIL_BASE = ROWS * 128
TAIL_ELEMS = N - TAIL_BASE
TAIL_VEC = TAIL_ELEMS // L
CAND_ROWS = NW * L // 128
BE = BR * 128
BIG_I32 = 1 << 30

_LOG_P = (
    7.0376836292e-2, -1.1514610310e-1, 1.1676998740e-1,
    -1.2420140846e-1, 1.4249322787e-1, -1.6668057665e-1,
    2.0000714765e-1, -2.4999993993e-1, 3.3333331174e-1,
)
_LN2_HI = 0.693359375
_LN2_LO = -2.12194440e-4
_SQRT_HALF_BITS = 0x3F3504F3


def _log(u):
    bits = lax.bitcast_convert_type(u, jnp.int32)
    e = lax.shift_right_arithmetic(bits - _SQRT_HALF_BITS, 23)
    m = lax.bitcast_convert_type(bits - lax.shift_left(e, 23), jnp.float32)
    f = m - 1.0
    ef = e.astype(jnp.float32)
    z = f * f
    poly = jnp.float32(_LOG_P[0])
    for c in _LOG_P[1:]:
        poly = poly * f + c
    y = poly * f * z + ef * _LN2_LO - 0.5 * z
    return (f + y) + ef * _LN2_HI


def _score(u, p):
    return _log(u) * jnp.exp(-p)


def _upd(xb, ib, x, idx):
    take = x > xb
    return jnp.where(take, x, xb), jnp.where(take, idx, ib)


def _merge(xa, ia, xb, ibv):
    take = (xb > xa) | ((xb == xa) & (ibv < ia))
    return jnp.where(take, xb, xa), jnp.where(take, ibv, ia)


def _sc_body(p_hbm, u_hbm, x_out, i_out, u_v, p_v, x_s, i_s, dsem):
    c = lax.axis_index("c")
    s = lax.axis_index("s")
    wid = c * NS + s
    base = SC_START + wid * CHUNK
    copies = [
        pltpu.async_copy(u_hbm.at[pl.ds(base, CHUNK)],
                         u_v.at[pl.ds(0, CHUNK)], dsem),
        pltpu.async_copy(p_hbm.at[pl.ds(base, CHUNK)],
                         p_v.at[pl.ds(0, CHUNK)], dsem),
        pltpu.async_copy(u_hbm.at[pl.ds(TAIL_BASE, TAIL_ELEMS)],
                         u_v.at[pl.ds(CHUNK, TAIL_ELEMS)], dsem),
        pltpu.async_copy(p_hbm.at[pl.ds(TAIL_BASE, TAIL_ELEMS)],
                         p_v.at[pl.ds(CHUNK, TAIL_ELEMS)], dsem),
    ]
    for cpy in copies:
        cpy.wait()

    iota = lax.iota(jnp.int32, L)
    neg_inf = jnp.full((L,), -jnp.inf, jnp.float32)
    big = jnp.full((L,), BIG_I32, jnp.int32)

    def step(i, carry):
        st = list(carry)
        off = i * (UNROLL * L)
        for j in range(UNROLL):
            o = off + j * L
            x = _score(u_v[pl.ds(o, L)], p_v[pl.ds(o, L)])
            st[2 * j], st[2 * j + 1] = _upd(st[2 * j], st[2 * j + 1], x,
                                            iota + (base + o))
        return tuple(st)

    st = lax.fori_loop(0, NMAIN, step, (neg_inf, big) * UNROLL)
    st = list(st)

    for k in range(TAIL_VEC):
        o = CHUNK + k * L
        x = _score(u_v[pl.ds(o, L)], p_v[pl.ds(o, L)])
        j = k % UNROLL
        st[2 * j], st[2 * j + 1] = _upd(st[2 * j], st[2 * j + 1], x,
                                        iota + (TAIL_BASE + k * L))

    xb, ib = st[0], st[1]
    for j in range(1, UNROLL):
        xb, ib = _merge(xb, ib, st[2 * j], st[2 * j + 1])

    x_s[...] = xb
    i_s[...] = ib
    co1 = pltpu.async_copy(x_s, x_out.at[pl.ds(wid * L, L)], dsem)
    co2 = pltpu.async_copy(i_s, i_out.at[pl.ds(wid * L, L)], dsem)
    co1.wait()
    co2.wait()


_sc_mesh = plsc.VectorSubcoreMesh(core_axis_name="c", subcore_axis_name="s",
                                  num_cores=NC, num_subcores=NS)

_sc_stage = pl.kernel(
    _sc_body,
    out_type=(jax.ShapeDtypeStruct((NW * L,), jnp.float32),
              jax.ShapeDtypeStruct((NW * L,), jnp.int32)),
    mesh=_sc_mesh,
    scratch_types=[
        pltpu.VMEM((CHUNK + TAIL_ELEMS,), jnp.float32),
        pltpu.VMEM((CHUNK + TAIL_ELEMS,), jnp.float32),
        pltpu.VMEM((L,), jnp.float32),
        pltpu.VMEM((L,), jnp.int32),
        pltpu.SemaphoreType.DMA,
    ],
)


def _tc_body(p_hbm, u_hbm, xc_ref, ic_ref, out_ref,
             pb0, ub0, pb1, ub1, acc_x, acc_b, sem0, sem1):
    def copies(i, pb, ub, sem):
        return (pltpu.make_async_copy(p_hbm.at[pl.ds(i * BE, BE)], pb, sem),
                pltpu.make_async_copy(u_hbm.at[pl.ds(i * BE, BE)], ub, sem))

    for cpy in copies(0, pb0, ub0, sem0):
        cpy.start()

    def phase(i, pb, ub, sem, npb, nub, nsem):
        @pl.when(i + 1 < GRID)
        def _prefetch():
            for cpy in copies(i + 1, npb, nub, nsem):
                cpy.start()

        for cpy in copies(i, pb, ub, sem):
            cpy.wait()
        xw = jnp.log(ub[...]) * jnp.exp(-pb[...])

        @pl.when(i == 0)
        def _init():
            acc_x[...] = xw
            acc_b[...] = jnp.zeros((BE,), jnp.int32)

        @pl.when(i > 0)
        def _acc():
            take = xw > acc_x[...]
            acc_x[...] = jnp.where(take, xw, acc_x[...])
            acc_b[...] = jnp.where(take,
                                   jnp.broadcast_to(i, (BE,)),
                                   acc_b[...])

    def body(i, carry):
        @pl.when(lax.rem(i, 2) == 0)
        def _even():
            phase(i, pb0, ub0, sem0, pb1, ub1, sem1)

        @pl.when(lax.rem(i, 2) == 1)
        def _odd():
            phase(i, pb1, ub1, sem1, pb0, ub0, sem0)

        return carry

    lax.fori_loop(0, GRID, body, 0)

    ax = acc_x[...]
    ab = acc_b[...]
    pos = lax.broadcasted_iota(jnp.int32, (BE,), 0)
    gidx = ab * BE + pos
    m_tc = jnp.max(ax)
    i_tc = jnp.min(jnp.where(ax == m_tc, gidx, BIG_I32))
    xc = xc_ref[...]
    ic = ic_ref[...]
    m_sc = jnp.max(xc)
    i_sc = jnp.min(jnp.where(xc == m_sc, ic, BIG_I32))
    take_sc = (m_sc > m_tc) | ((m_sc == m_tc) & (i_sc < i_tc))
    out_ref[0, 0] = jnp.where(take_sc, i_sc, i_tc)


_tc_stage = pl.pallas_call(
    _tc_body,
    in_specs=[
        pl.BlockSpec(memory_space=pl.ANY),
        pl.BlockSpec(memory_space=pl.ANY),
        pl.BlockSpec((CAND_ROWS, 128), lambda: (0, 0)),
        pl.BlockSpec((CAND_ROWS, 128), lambda: (0, 0)),
    ],
    out_specs=pl.BlockSpec(memory_space=pltpu.SMEM),
    out_shape=jax.ShapeDtypeStruct((1, 1), jnp.int32),
    scratch_shapes=[
        pltpu.VMEM((BE,), jnp.float32),
        pltpu.VMEM((BE,), jnp.float32),
        pltpu.VMEM((BE,), jnp.float32),
        pltpu.VMEM((BE,), jnp.float32),
        pltpu.VMEM((BE,), jnp.float32),
        pltpu.VMEM((BE,), jnp.int32),
        pltpu.SemaphoreType.DMA,
        pltpu.SemaphoreType.DMA,
    ],
)


def kernel(p, u):
    x_cand, i_cand = _sc_stage(p, u)
    ans = _tc_stage(p, u,
                    x_cand.reshape(CAND_ROWS, 128),
                    i_cand.reshape(CAND_ROWS, 128))
    return ans[0, 0]
